# baseline (speedup 1.0000x reference)
import functools

import jax
import jax.numpy as jnp
from jax import lax
from jax.experimental import pallas as pl
from jax.experimental.pallas import tpu as pltpu
from jax.experimental.pallas import tpu_sc as plsc

N_NODES = 10000
N_EDGES = 320000
D = 128

NC = 2
NS = 16
NW = NC * NS

N_PAD = 10240
TRASH = 10200
ROWS_PER_TILE = N_PAD // NS
CHUNK = 32
KB = 16
NKB0 = 26
NKB1 = 14
NCHUNK = NKB0 * KB
E_PAD = NS * (NKB0 + NKB1) * KB * CHUNK
NBUF = 3
CSTG = 64


def _sc_aggregate_body(x_hbm, src_hbm, dst_hbm, zrows_hbm, zc_hbm, ones_hbm,
                       agg_out, cnt_out,
                       src_v, dst_v, rows_a, rows_b, rows_c, ones_v,
                       agg_sh, cnt_sh,
                       gsem_a, gsem_b, gsem_c, ssem_a, ssem_b, ssem_c, csem):
    c = lax.axis_index("c")
    s = lax.axis_index("s")
    wid = c * NS + s
    nkb = jnp.where(c == 0, NKB0, NKB1)

    base = s * ROWS_PER_TILE
    pltpu.sync_copy(zrows_hbm, rows_a)
    pltpu.sync_copy(zc_hbm, ones_v)
    nfull = ROWS_PER_TILE // CHUNK
    for j in range(nfull):
        pltpu.sync_copy(rows_a, agg_sh.at[pl.ds(base + j * CHUNK, CHUNK)])
    for j in range(ROWS_PER_TILE // CSTG):
        pltpu.sync_copy(ones_v, cnt_sh.at[pl.ds(base + j * CSTG, CSTG)])
    pltpu.sync_copy(ones_hbm, ones_v)
    plsc.subcore_barrier()

    rows = (rows_a, rows_b, rows_c)
    gsem = (gsem_a, gsem_b, gsem_c)
    ssem = (ssem_a, ssem_b, ssem_c)

    def block_body(kb, carry):
        k0 = pl.multiple_of(kb * KB, KB)
        pltpu.sync_copy(src_hbm.at[wid, pl.ds(k0, KB)], src_v)
        pltpu.sync_copy(dst_hbm.at[wid, pl.ds(k0, KB)], dst_v)

        g = [None] * NBUF
        sc = [None] * NBUF
        cprev = None
        g[0] = pltpu.async_copy(x_hbm.at[src_v.at[0]], rows[0], gsem[0])
        g[1] = pltpu.async_copy(x_hbm.at[src_v.at[1]], rows[1], gsem[1])
        for i in range(KB):
            b = i % NBUF
            g[b].wait()
            sc[b] = pltpu.async_copy(
                rows[b], agg_sh.at[dst_v.at[i]], ssem[b], add=True)
            if i + 2 < KB:
                b2 = (i + 2) % NBUF
                if sc[b2] is not None:
                    sc[b2].wait()
                g[b2] = pltpu.async_copy(
                    x_hbm.at[src_v.at[i + 2]], rows[b2], gsem[b2])
            cnew = pltpu.async_copy(
                ones_v.at[pl.ds(0, CHUNK)], cnt_sh.at[dst_v.at[i]], csem,
                add=True)
            if cprev is not None:
                cprev.wait()
            cprev = cnew
        for b in range(NBUF):
            sc[b].wait()
        cprev.wait()
        return carry

    lax.fori_loop(0, nkb, block_body, 0)
    plsc.subcore_barrier()

    out_base = c * N_PAD + base
    for j in range(nfull):
        pltpu.sync_copy(agg_sh.at[pl.ds(base + j * CHUNK, CHUNK)], rows_a)
        pltpu.sync_copy(rows_a, agg_out.at[pl.ds(out_base + j * CHUNK, CHUNK)])
    for j in range(ROWS_PER_TILE // CSTG):
        pltpu.sync_copy(cnt_sh.at[pl.ds(base + j * CSTG, CSTG)], ones_v)
        pltpu.sync_copy(ones_v, cnt_out.at[pl.ds(out_base + j * CSTG, CSTG)])


_sc_aggregate = functools.partial(
    pl.kernel,
    mesh=plsc.VectorSubcoreMesh(core_axis_name="c", subcore_axis_name="s"),
    compiler_params=pltpu.CompilerParams(use_tc_tiling_on_sc=False),
    out_type=[
        jax.ShapeDtypeStruct((NC * N_PAD, D), jnp.float32),
        jax.ShapeDtypeStruct((NC * N_PAD,), jnp.float32),
    ],
    scratch_types=[
        pltpu.VMEM((KB, CHUNK), jnp.int32),
        pltpu.VMEM((KB, CHUNK), jnp.int32),
        pltpu.VMEM((CHUNK, D), jnp.float32),
        pltpu.VMEM((CHUNK, D), jnp.float32),
        pltpu.VMEM((CHUNK, D), jnp.float32),
        pltpu.VMEM((CSTG,), jnp.float32),
        pltpu.VMEM_SHARED((N_PAD, D), jnp.float32),
        pltpu.VMEM_SHARED((N_PAD,), jnp.float32),
        pltpu.SemaphoreType.DMA,
        pltpu.SemaphoreType.DMA,
        pltpu.SemaphoreType.DMA,
        pltpu.SemaphoreType.DMA,
        pltpu.SemaphoreType.DMA,
        pltpu.SemaphoreType.DMA,
        pltpu.SemaphoreType.DMA,
    ],
)(_sc_aggregate_body)


def _tc_block_body(agg_ref, cnt_ref, x_ref, wl_ref, wr_ref, b_ref, g_ref,
                   be_ref, o_ref):
    a = agg_ref[0] + agg_ref[1]
    cn = cnt_ref[0] + cnt_ref[1]
    h = a * (1.0 / jnp.maximum(cn, 1.0))
    xb = x_ref[...]
    f = (jnp.dot(h, wl_ref[...], preferred_element_type=jnp.float32)
         + jnp.dot(xb, wr_ref[...], preferred_element_type=jnp.float32)
         + b_ref[...])
    f = 0.5 * f * (1.0 + lax.erf(f * 0.7071067811865476))
    mean = jnp.mean(f, axis=1, keepdims=True)
    d = f - mean
    var = jnp.mean(d * d, axis=1, keepdims=True)
    f = d * lax.rsqrt(var + 1e-5) * g_ref[...] + be_ref[...]
    o_ref[...] = f + xb


def _tc_tail(agg, cnt, x, wlT, wrT, b, g, be):
    R = 1000
    grid = (N_NODES // R,)
    return pl.pallas_call(
        _tc_block_body,
        grid=grid,
        in_specs=[
            pl.BlockSpec((NC, R, D), lambda i: (0, i, 0)),
            pl.BlockSpec((NC, R, 1), lambda i: (0, i, 0)),
            pl.BlockSpec((R, D), lambda i: (i, 0)),
            pl.BlockSpec((D, D), lambda i: (0, 0)),
            pl.BlockSpec((D, D), lambda i: (0, 0)),
            pl.BlockSpec((1, D), lambda i: (0, 0)),
            pl.BlockSpec((1, D), lambda i: (0, 0)),
            pl.BlockSpec((1, D), lambda i: (0, 0)),
        ],
        out_specs=pl.BlockSpec((R, D), lambda i: (i, 0)),
        out_shape=jax.ShapeDtypeStruct((N_NODES, D), jnp.float32),
    )(agg, cnt, x, wlT, wrT, b, g, be)


def kernel(x, edge_index, W_l, b_l, W_r, gamma, beta):
    npad = E_PAD - N_EDGES
    pad_src = (jnp.arange(npad, dtype=jnp.int32) * 13) % N_NODES
    E0 = NS * NKB0 * KB * CHUNK

    def _split(flat):
        part0 = flat[:E0].reshape(NS, NKB0 * KB, CHUNK)
        part1 = flat[E0:].reshape(NS, NKB1 * KB, CHUNK)
        part1 = jnp.pad(part1, ((0, 0), (0, (NKB0 - NKB1) * KB), (0, 0)))
        return jnp.concatenate([part0, part1], axis=0)

    src = _split(jnp.concatenate([edge_index[0].astype(jnp.int32), pad_src]))
    dst = _split(jnp.concatenate(
        [edge_index[1].astype(jnp.int32), jnp.full((npad,), TRASH, jnp.int32)]))
    zrows = jnp.zeros((CHUNK, D), jnp.float32)
    zc = jnp.zeros((CSTG,), jnp.float32)
    ones = jnp.ones((CSTG,), jnp.float32)
    agg_flat, cnt_flat = _sc_aggregate(x, src, dst, zrows, zc, ones)
    agg = agg_flat.reshape(NC, N_PAD, D)
    cnt = cnt_flat.reshape(NC, N_PAD, 1)
    return _tc_tail(agg, cnt, x, W_l.T, W_r.T,
                    b_l.reshape(1, D), gamma.reshape(1, D), beta.reshape(1, D))


# speedup vs baseline: 1.1051x; 1.1051x over previous
import functools

import jax
import jax.numpy as jnp
from jax import lax
from jax.experimental import pallas as pl
from jax.experimental.pallas import tpu as pltpu
from jax.experimental.pallas import tpu_sc as plsc

N_NODES = 10000
N_EDGES = 320000
D = 128

NC = 2
NS = 16
NW = NC * NS

N_PAD = 10240
TRASH = 10200
ROWS_PER_TILE = N_PAD // NS
CHUNK = 32
KB = 16
NKB0 = 26
NKB1 = 14
NCHUNK = NKB0 * KB
E_PAD = NS * (NKB0 + NKB1) * KB * CHUNK
NBUF = 3
CSTG = 64


def _sc_aggregate_body(x_hbm, src_hbm, dst_hbm, zrows_hbm, zc_hbm, ones_hbm,
                       agg_out, cnt_out,
                       src_a, dst_a, src_b, dst_b, rows_a, rows_b, rows_c,
                       ones_v, agg_sh, cnt_sh,
                       gsem_a, gsem_b, gsem_c, ssem_a, ssem_b, ssem_c, csem,
                       isem_a, isem_b):
    c = lax.axis_index("c")
    s = lax.axis_index("s")
    wid = c * NS + s
    npair = jnp.where(c == 0, NKB0 // 2, NKB1 // 2)

    base = s * ROWS_PER_TILE
    pltpu.sync_copy(zrows_hbm, rows_a)
    pltpu.sync_copy(zc_hbm, ones_v)
    nfull = ROWS_PER_TILE // CHUNK
    for j in range(nfull):
        pltpu.sync_copy(rows_a, agg_sh.at[pl.ds(base + j * CHUNK, CHUNK)])
    for j in range(ROWS_PER_TILE // CSTG):
        pltpu.sync_copy(ones_v, cnt_sh.at[pl.ds(base + j * CSTG, CSTG)])
    pltpu.sync_copy(ones_hbm, ones_v)
    plsc.subcore_barrier()

    rows = (rows_a, rows_b, rows_c)
    gsem = (gsem_a, gsem_b, gsem_c)
    ssem = (ssem_a, ssem_b, ssem_c)

    def _stage(kb, sv, dv, isem):
        k0 = pl.multiple_of(kb * KB, KB)
        ds_ = pltpu.async_copy(src_hbm.at[wid, pl.ds(k0, KB)], sv, isem)
        dd_ = pltpu.async_copy(dst_hbm.at[wid, pl.ds(k0, KB)], dv, isem)
        return ds_, dd_

    _stage(0, src_a, dst_a, isem_a)

    def pair_body(p, carry):
        pltpu.make_async_copy(src_hbm.at[wid, pl.ds(0, KB)], src_a,
                              isem_a).wait()
        pltpu.make_async_copy(dst_hbm.at[wid, pl.ds(0, KB)], dst_a,
                              isem_a).wait()
        _stage(2 * p + 1, src_b, dst_b, isem_b)

        g = [None] * NBUF
        sc = [None] * NBUF
        cprev = None
        g[0] = pltpu.async_copy(x_hbm.at[src_a.at[0]], rows[0], gsem[0])
        g[1] = pltpu.async_copy(x_hbm.at[src_a.at[1]], rows[1], gsem[1])
        for i in range(2 * KB):
            sv, dv = (src_a, dst_a) if i < KB else (src_b, dst_b)
            ii = i % KB
            b = i % NBUF
            g[b].wait()
            sc[b] = pltpu.async_copy(
                rows[b], agg_sh.at[dv.at[ii]], ssem[b], add=True)
            if i == KB - 1:
                pltpu.make_async_copy(src_hbm.at[wid, pl.ds(0, KB)], src_b,
                                      isem_b).wait()
                pltpu.make_async_copy(dst_hbm.at[wid, pl.ds(0, KB)], dst_b,
                                      isem_b).wait()
                _stage(jnp.minimum(2 * p + 2, 2 * npair - 1), src_a, dst_a,
                       isem_a)
            if i + 2 < 2 * KB:
                i2 = i + 2
                sv2 = src_a if i2 < KB else src_b
                b2 = i2 % NBUF
                if sc[b2] is not None:
                    sc[b2].wait()
                g[b2] = pltpu.async_copy(
                    x_hbm.at[sv2.at[i2 % KB]], rows[b2], gsem[b2])
            cnew = pltpu.async_copy(
                ones_v.at[pl.ds(0, CHUNK)], cnt_sh.at[dv.at[ii]], csem,
                add=True)
            if cprev is not None:
                cprev.wait()
            cprev = cnew
        for b in range(NBUF):
            sc[b].wait()
        cprev.wait()
        return carry

    lax.fori_loop(0, npair, pair_body, 0)
    pltpu.make_async_copy(src_hbm.at[wid, pl.ds(0, KB)], src_a, isem_a).wait()
    pltpu.make_async_copy(dst_hbm.at[wid, pl.ds(0, KB)], dst_a, isem_a).wait()
    plsc.subcore_barrier()

    out_base = c * N_PAD + base
    for j in range(nfull):
        pltpu.sync_copy(agg_sh.at[pl.ds(base + j * CHUNK, CHUNK)], rows_a)
        pltpu.sync_copy(rows_a, agg_out.at[pl.ds(out_base + j * CHUNK, CHUNK)])
    for j in range(ROWS_PER_TILE // CSTG):
        pltpu.sync_copy(cnt_sh.at[pl.ds(base + j * CSTG, CSTG)], ones_v)
        pltpu.sync_copy(ones_v, cnt_out.at[pl.ds(out_base + j * CSTG, CSTG)])


_sc_aggregate = functools.partial(
    pl.kernel,
    mesh=plsc.VectorSubcoreMesh(core_axis_name="c", subcore_axis_name="s"),
    compiler_params=pltpu.CompilerParams(use_tc_tiling_on_sc=False),
    out_type=[
        jax.ShapeDtypeStruct((NC * N_PAD, D), jnp.float32),
        jax.ShapeDtypeStruct((NC * N_PAD,), jnp.float32),
    ],
    scratch_types=[
        pltpu.VMEM((KB, CHUNK), jnp.int32),
        pltpu.VMEM((KB, CHUNK), jnp.int32),
        pltpu.VMEM((KB, CHUNK), jnp.int32),
        pltpu.VMEM((KB, CHUNK), jnp.int32),
        pltpu.VMEM((CHUNK, D), jnp.float32),
        pltpu.VMEM((CHUNK, D), jnp.float32),
        pltpu.VMEM((CHUNK, D), jnp.float32),
        pltpu.VMEM((CSTG,), jnp.float32),
        pltpu.VMEM_SHARED((N_PAD, D), jnp.float32),
        pltpu.VMEM_SHARED((N_PAD,), jnp.float32),
        pltpu.SemaphoreType.DMA,
        pltpu.SemaphoreType.DMA,
        pltpu.SemaphoreType.DMA,
        pltpu.SemaphoreType.DMA,
        pltpu.SemaphoreType.DMA,
        pltpu.SemaphoreType.DMA,
        pltpu.SemaphoreType.DMA,
        pltpu.SemaphoreType.DMA,
        pltpu.SemaphoreType.DMA,
    ],
)(_sc_aggregate_body)


def _tc_block_body(agg_ref, cnt_ref, x_ref, wl_ref, wr_ref, b_ref, g_ref,
                   be_ref, o_ref):
    a = agg_ref[0] + agg_ref[1]
    cn = cnt_ref[0] + cnt_ref[1]
    h = a * (1.0 / jnp.maximum(cn, 1.0))
    xb = x_ref[...]
    f = (jnp.dot(h, wl_ref[...], preferred_element_type=jnp.float32)
         + jnp.dot(xb, wr_ref[...], preferred_element_type=jnp.float32)
         + b_ref[...])
    f = 0.5 * f * (1.0 + lax.erf(f * 0.7071067811865476))
    mean = jnp.mean(f, axis=1, keepdims=True)
    d = f - mean
    var = jnp.mean(d * d, axis=1, keepdims=True)
    f = d * lax.rsqrt(var + 1e-5) * g_ref[...] + be_ref[...]
    o_ref[...] = f + xb


def _tc_tail(agg, cnt, x, wlT, wrT, b, g, be):
    R = 1000
    grid = (N_NODES // R,)
    return pl.pallas_call(
        _tc_block_body,
        grid=grid,
        in_specs=[
            pl.BlockSpec((NC, R, D), lambda i: (0, i, 0)),
            pl.BlockSpec((NC, R, 1), lambda i: (0, i, 0)),
            pl.BlockSpec((R, D), lambda i: (i, 0)),
            pl.BlockSpec((D, D), lambda i: (0, 0)),
            pl.BlockSpec((D, D), lambda i: (0, 0)),
            pl.BlockSpec((1, D), lambda i: (0, 0)),
            pl.BlockSpec((1, D), lambda i: (0, 0)),
            pl.BlockSpec((1, D), lambda i: (0, 0)),
        ],
        out_specs=pl.BlockSpec((R, D), lambda i: (i, 0)),
        out_shape=jax.ShapeDtypeStruct((N_NODES, D), jnp.float32),
    )(agg, cnt, x, wlT, wrT, b, g, be)


def kernel(x, edge_index, W_l, b_l, W_r, gamma, beta):
    npad = E_PAD - N_EDGES
    pad_src = (jnp.arange(npad, dtype=jnp.int32) * 13) % N_NODES
    E0 = NS * NKB0 * KB * CHUNK

    def _split(flat):
        part0 = flat[:E0].reshape(NS, NKB0 * KB, CHUNK)
        part1 = flat[E0:].reshape(NS, NKB1 * KB, CHUNK)
        part1 = jnp.pad(part1, ((0, 0), (0, (NKB0 - NKB1) * KB), (0, 0)))
        return jnp.concatenate([part0, part1], axis=0)

    src = _split(jnp.concatenate([edge_index[0].astype(jnp.int32), pad_src]))
    dst = _split(jnp.concatenate(
        [edge_index[1].astype(jnp.int32), jnp.full((npad,), TRASH, jnp.int32)]))
    zrows = jnp.zeros((CHUNK, D), jnp.float32)
    zc = jnp.zeros((CSTG,), jnp.float32)
    ones = jnp.ones((CSTG,), jnp.float32)
    agg_flat, cnt_flat = _sc_aggregate(x, src, dst, zrows, zc, ones)
    agg = agg_flat.reshape(NC, N_PAD, D)
    cnt = cnt_flat.reshape(NC, N_PAD, 1)
    return _tc_tail(agg, cnt, x, W_l.T, W_r.T,
                    b_l.reshape(1, D), gamma.reshape(1, D), beta.reshape(1, D))


# speedup vs baseline: 1.4484x; 1.3106x over previous
import functools

import jax
import jax.numpy as jnp
from jax import lax
from jax.experimental import pallas as pl
from jax.experimental.pallas import tpu as pltpu
from jax.experimental.pallas import tpu_sc as plsc

N_NODES = 10000
N_EDGES = 320000
D = 128

NC = 2
NS = 16
NW = NC * NS

N_PAD = 10240
TRASH = 10200
ROWS_PER_TILE = N_PAD // NS
CHUNK = 32
KB = 16
NKB = 20
NPAIR = NKB // 2
NCHUNK = NKB * KB
E_PAD = NW * NCHUNK * CHUNK
NBUF = 3
CSTG = 64


def _sc_aggregate_body(x_hbm, src_hbm, dst_hbm, zrows_hbm, zc_hbm, ones_hbm,
                       agg_out, cnt_out,
                       src_a, dst_a, src_b, dst_b, rows_a, rows_b, rows_c,
                       ones_v, agg_sh, cnt_sh,
                       gsem_a, gsem_b, gsem_c, ssem_a, ssem_b, ssem_c, csem,
                       isem_a, isem_b):
    c = lax.axis_index("c")
    s = lax.axis_index("s")
    wid = c * NS + s

    base = s * ROWS_PER_TILE
    pltpu.sync_copy(zrows_hbm, rows_a)
    pltpu.sync_copy(zc_hbm, ones_v)
    nfull = ROWS_PER_TILE // CHUNK
    for j in range(nfull):
        pltpu.sync_copy(rows_a, agg_sh.at[pl.ds(base + j * CHUNK, CHUNK)])
    for j in range(ROWS_PER_TILE // CSTG):
        pltpu.sync_copy(ones_v, cnt_sh.at[pl.ds(base + j * CSTG, CSTG)])
    pltpu.sync_copy(ones_hbm, ones_v)
    plsc.subcore_barrier()

    rows = (rows_a, rows_b, rows_c)
    gsem = (gsem_a, gsem_b, gsem_c)
    ssem = (ssem_a, ssem_b, ssem_c)

    def _stage(kb, sv, dv, isem):
        k0 = pl.multiple_of(kb * KB, KB)
        ds_ = pltpu.async_copy(src_hbm.at[wid, pl.ds(k0, KB)], sv, isem)
        dd_ = pltpu.async_copy(dst_hbm.at[wid, pl.ds(k0, KB)], dv, isem)
        return ds_, dd_

    _stage(0, src_a, dst_a, isem_a)

    def pair_body(p, carry):
        pltpu.make_async_copy(src_hbm.at[wid, pl.ds(0, KB)], src_a,
                              isem_a).wait()
        pltpu.make_async_copy(dst_hbm.at[wid, pl.ds(0, KB)], dst_a,
                              isem_a).wait()
        _stage(2 * p + 1, src_b, dst_b, isem_b)

        g = [None] * NBUF
        sc = [None] * NBUF
        cprev = None
        g[0] = pltpu.async_copy(x_hbm.at[src_a.at[0]], rows[0], gsem[0])
        g[1] = pltpu.async_copy(x_hbm.at[src_a.at[1]], rows[1], gsem[1])
        for i in range(2 * KB):
            sv, dv = (src_a, dst_a) if i < KB else (src_b, dst_b)
            ii = i % KB
            b = i % NBUF
            g[b].wait()
            sc[b] = pltpu.async_copy(
                rows[b], agg_sh.at[dv.at[ii]], ssem[b], add=True)
            if i == KB - 1:
                pltpu.make_async_copy(src_hbm.at[wid, pl.ds(0, KB)], src_b,
                                      isem_b).wait()
                pltpu.make_async_copy(dst_hbm.at[wid, pl.ds(0, KB)], dst_b,
                                      isem_b).wait()
                _stage(jnp.minimum(2 * p + 2, NKB - 1), src_a, dst_a,
                       isem_a)
            if i + 2 < 2 * KB:
                i2 = i + 2
                sv2 = src_a if i2 < KB else src_b
                b2 = i2 % NBUF
                if sc[b2] is not None:
                    sc[b2].wait()
                g[b2] = pltpu.async_copy(
                    x_hbm.at[sv2.at[i2 % KB]], rows[b2], gsem[b2])
            cnew = pltpu.async_copy(
                ones_v.at[pl.ds(0, CHUNK)], cnt_sh.at[dv.at[ii]], csem,
                add=True)
            if cprev is not None:
                cprev.wait()
            cprev = cnew
        for b in range(NBUF):
            sc[b].wait()
        cprev.wait()
        return carry

    lax.fori_loop(0, NPAIR, pair_body, 0)
    pltpu.make_async_copy(src_hbm.at[wid, pl.ds(0, KB)], src_a, isem_a).wait()
    pltpu.make_async_copy(dst_hbm.at[wid, pl.ds(0, KB)], dst_a, isem_a).wait()
    plsc.subcore_barrier()

    out_base = c * N_PAD + base
    for j in range(nfull):
        pltpu.sync_copy(agg_sh.at[pl.ds(base + j * CHUNK, CHUNK)], rows_a)
        pltpu.sync_copy(rows_a, agg_out.at[pl.ds(out_base + j * CHUNK, CHUNK)])
    for j in range(ROWS_PER_TILE // CSTG):
        pltpu.sync_copy(cnt_sh.at[pl.ds(base + j * CSTG, CSTG)], ones_v)
        pltpu.sync_copy(ones_v, cnt_out.at[pl.ds(out_base + j * CSTG, CSTG)])


_sc_aggregate = functools.partial(
    pl.kernel,
    mesh=plsc.VectorSubcoreMesh(core_axis_name="c", subcore_axis_name="s"),
    out_type=[
        jax.ShapeDtypeStruct((NC * N_PAD, D), jnp.float32),
        jax.ShapeDtypeStruct((NC * N_PAD,), jnp.float32),
    ],
    scratch_types=[
        pltpu.VMEM((KB, CHUNK), jnp.int32),
        pltpu.VMEM((KB, CHUNK), jnp.int32),
        pltpu.VMEM((KB, CHUNK), jnp.int32),
        pltpu.VMEM((KB, CHUNK), jnp.int32),
        pltpu.VMEM((CHUNK, D), jnp.float32),
        pltpu.VMEM((CHUNK, D), jnp.float32),
        pltpu.VMEM((CHUNK, D), jnp.float32),
        pltpu.VMEM((CSTG,), jnp.float32),
        pltpu.VMEM_SHARED((N_PAD, D), jnp.float32),
        pltpu.VMEM_SHARED((N_PAD,), jnp.float32),
        pltpu.SemaphoreType.DMA,
        pltpu.SemaphoreType.DMA,
        pltpu.SemaphoreType.DMA,
        pltpu.SemaphoreType.DMA,
        pltpu.SemaphoreType.DMA,
        pltpu.SemaphoreType.DMA,
        pltpu.SemaphoreType.DMA,
        pltpu.SemaphoreType.DMA,
        pltpu.SemaphoreType.DMA,
    ],
)(_sc_aggregate_body)


def _tc_block_body(agg_ref, cnt_ref, x_ref, wl_ref, wr_ref, b_ref, g_ref,
                   be_ref, o_ref):
    a = agg_ref[0] + agg_ref[1]
    cn = cnt_ref[0] + cnt_ref[1]
    h = a * (1.0 / jnp.maximum(cn, 1.0))
    xb = x_ref[...]
    f = (jnp.dot(h, wl_ref[...], preferred_element_type=jnp.float32)
         + jnp.dot(xb, wr_ref[...], preferred_element_type=jnp.float32)
         + b_ref[...])
    f = 0.5 * f * (1.0 + lax.erf(f * 0.7071067811865476))
    mean = jnp.mean(f, axis=1, keepdims=True)
    d = f - mean
    var = jnp.mean(d * d, axis=1, keepdims=True)
    f = d * lax.rsqrt(var + 1e-5) * g_ref[...] + be_ref[...]
    o_ref[...] = f + xb


def _tc_tail(agg, cnt, x, wlT, wrT, b, g, be):
    R = 1000
    grid = (N_NODES // R,)
    return pl.pallas_call(
        _tc_block_body,
        grid=grid,
        in_specs=[
            pl.BlockSpec((NC, R, D), lambda i: (0, i, 0)),
            pl.BlockSpec((NC, R, 1), lambda i: (0, i, 0)),
            pl.BlockSpec((R, D), lambda i: (i, 0)),
            pl.BlockSpec((D, D), lambda i: (0, 0)),
            pl.BlockSpec((D, D), lambda i: (0, 0)),
            pl.BlockSpec((1, D), lambda i: (0, 0)),
            pl.BlockSpec((1, D), lambda i: (0, 0)),
            pl.BlockSpec((1, D), lambda i: (0, 0)),
        ],
        out_specs=pl.BlockSpec((R, D), lambda i: (i, 0)),
        out_shape=jax.ShapeDtypeStruct((N_NODES, D), jnp.float32),
    )(agg, cnt, x, wlT, wrT, b, g, be)


def kernel(x, edge_index, W_l, b_l, W_r, gamma, beta):
    npad = E_PAD - N_EDGES
    pad_src = (jnp.arange(npad, dtype=jnp.int32) * 13) % N_NODES
    src = jnp.concatenate([edge_index[0].astype(jnp.int32), pad_src]
                          ).reshape(NW, NCHUNK, CHUNK)
    dst = jnp.concatenate(
        [edge_index[1].astype(jnp.int32), jnp.full((npad,), TRASH, jnp.int32)]
    ).reshape(NW, NCHUNK, CHUNK)
    zrows = jnp.zeros((CHUNK, D), jnp.float32)
    zc = jnp.zeros((CSTG,), jnp.float32)
    ones = jnp.ones((CSTG,), jnp.float32)
    agg_flat, cnt_flat = _sc_aggregate(x, src, dst, zrows, zc, ones)
    agg = agg_flat.reshape(NC, N_PAD, D)
    cnt = cnt_flat.reshape(NC, N_PAD, 1)
    return _tc_tail(agg, cnt, x, W_l.T, W_r.T,
                    b_l.reshape(1, D), gamma.reshape(1, D), beta.reshape(1, D))
